# baseline (speedup 1.0000x reference)
import functools

import jax
import jax.numpy as jnp
from jax.experimental import pallas as pl
from jax.experimental.pallas import tpu as pltpu

HIDDEN = 4096
NUM_EXPERTS = 64
TOP_K = 8
NEG_INF = float("-inf")
NS = 4


def _topk_softmax(logits):
    work = logits.T
    eid = jax.lax.broadcasted_iota(jnp.int32, work.shape, 0).astype(jnp.float32)
    vals = []
    idxs = []
    for _ in range(TOP_K):
        m = jnp.max(work, axis=0, keepdims=True)
        idx = jnp.min(
            jnp.where(work == m, eid, float(NUM_EXPERTS)), axis=0, keepdims=True
        )
        vals.append(m)
        idxs.append(idx)
        work = jnp.where(eid == idx, NEG_INF, work)
    v = jnp.concatenate(vals, axis=0)
    i = jnp.concatenate(idxs, axis=0)
    e = jnp.exp(v - v[0:1, :])
    w = e / jnp.sum(e, axis=0, keepdims=True)
    return w.T, i.T


def _router_body(*refs):
    x_refs = refs[:NS]
    wt_ref, b_ref, logits_ref, w_ref, i_ref = refs[NS:]
    sub = x_refs[0].shape[0]
    for j in range(NS):
        logits = (
            jnp.dot(x_refs[j][...], wt_ref[...], preferred_element_type=jnp.float32)
            + b_ref[...]
        )
        rows = pl.ds(j * sub, sub)
        logits_ref[rows, :] = logits
        w, i = _topk_softmax(logits)
        w_ref[rows, :] = w
        i_ref[rows, :] = i.astype(jnp.int32)


@functools.partial(jax.jit, static_argnames=("block_tokens",))
def _router(hidden_states, W, b, block_tokens=1024):
    B, S, H = hidden_states.shape
    T = B * S
    x = hidden_states.reshape(T, H)
    wt = W.T
    b2 = b.reshape(1, NUM_EXPERTS)
    sub = block_tokens // NS

    grid = (T // block_tokens,)
    xspecs = [
        pl.BlockSpec(
            (sub, H), functools.partial(lambda j, t: (NS * t + j, 0), j)
        )
        for j in range(NS)
    ]
    logits, weights, indices = pl.pallas_call(
        _router_body,
        grid=grid,
        in_specs=xspecs
        + [
            pl.BlockSpec((H, NUM_EXPERTS), lambda t: (0, 0)),
            pl.BlockSpec((1, NUM_EXPERTS), lambda t: (0, 0)),
        ],
        out_specs=[
            pl.BlockSpec((block_tokens, NUM_EXPERTS), lambda t: (t, 0)),
            pl.BlockSpec((block_tokens, TOP_K), lambda t: (t, 0)),
            pl.BlockSpec((block_tokens, TOP_K), lambda t: (t, 0)),
        ],
        out_shape=[
            jax.ShapeDtypeStruct((T, NUM_EXPERTS), jnp.float32),
            jax.ShapeDtypeStruct((T, TOP_K), jnp.float32),
            jax.ShapeDtypeStruct((T, TOP_K), jnp.int32),
        ],
        compiler_params=pltpu.CompilerParams(
            dimension_semantics=("arbitrary",),
        ),
    )(*([x] * NS), wt, b2)

    return (
        weights.reshape(B, S, TOP_K),
        indices.reshape(B, S, TOP_K),
        logits.reshape(B, S, NUM_EXPERTS),
    )


def kernel(hidden_states, W, b):
    return _router(hidden_states, W, b)


# speedup vs baseline: 1.0009x; 1.0009x over previous
import functools

import jax
import jax.numpy as jnp
from jax.experimental import pallas as pl
from jax.experimental.pallas import tpu as pltpu

HIDDEN = 4096
NUM_EXPERTS = 64
TOP_K = 8
NEG_INF = float("-inf")
NS = 4


def _topk_softmax(logits):
    work = logits.T
    eid = jax.lax.broadcasted_iota(jnp.int32, work.shape, 0).astype(jnp.float32)
    vals = []
    idxs = []
    for _ in range(TOP_K):
        m = jnp.max(work, axis=0, keepdims=True)
        idx = jnp.min(
            jnp.where(work == m, eid, float(NUM_EXPERTS)), axis=0, keepdims=True
        )
        vals.append(m)
        idxs.append(idx)
        work = jnp.where(eid == idx, NEG_INF, work)
    v = jnp.concatenate(vals, axis=0)
    i = jnp.concatenate(idxs, axis=0)
    e = jnp.exp(v - v[0:1, :])
    w = e / jnp.sum(e, axis=0, keepdims=True)
    return w.T, i.T


def _router_body(*refs):
    x_refs = refs[:NS]
    wt_ref, b_ref, logits_ref, w_ref, i_ref = refs[NS:]
    sub = x_refs[0].shape[0]
    for j in range(NS):
        logits = (
            jnp.dot(x_refs[j][...], wt_ref[...], preferred_element_type=jnp.float32)
            + b_ref[...]
        )
        rows = pl.ds(j * sub, sub)
        logits_ref[rows, :] = logits
        w, i = _topk_softmax(logits)
        w_ref[rows, :] = w
        i_ref[rows, :] = i.astype(jnp.int32)


@functools.partial(jax.jit, static_argnames=("block_tokens",))
def _router(hidden_states, W, b, block_tokens=1024):
    B, S, H = hidden_states.shape
    T = B * S
    x = hidden_states.reshape(T, H)
    wt = W.T
    b2 = b.reshape(1, NUM_EXPERTS)
    sub = block_tokens // NS

    grid = (T // block_tokens,)
    xspecs = [
        pl.BlockSpec(
            (sub, H), functools.partial(lambda j, t: (NS * t + j, 0), j)
        )
        for j in range(NS)
    ]
    logits, weights, indices = pl.pallas_call(
        _router_body,
        grid=grid,
        in_specs=xspecs
        + [
            pl.BlockSpec((H, NUM_EXPERTS), lambda t: (0, 0)),
            pl.BlockSpec((1, NUM_EXPERTS), lambda t: (0, 0)),
        ],
        out_specs=[
            pl.BlockSpec((block_tokens, NUM_EXPERTS), lambda t: (t, 0)),
            pl.BlockSpec((block_tokens, TOP_K), lambda t: (t, 0)),
            pl.BlockSpec((block_tokens, TOP_K), lambda t: (t, 0)),
        ],
        out_shape=[
            jax.ShapeDtypeStruct((T, NUM_EXPERTS), jnp.float32),
            jax.ShapeDtypeStruct((T, TOP_K), jnp.float32),
            jax.ShapeDtypeStruct((T, TOP_K), jnp.int32),
        ],
        compiler_params=pltpu.CompilerParams(
            dimension_semantics=("parallel",),
        ),
    )(*([x] * NS), wt, b2)

    return (
        weights.reshape(B, S, TOP_K),
        indices.reshape(B, S, TOP_K),
        logits.reshape(B, S, NUM_EXPERTS),
    )


def kernel(hidden_states, W, b):
    return _router(hidden_states, W, b)
